# baseline (speedup 1.0000x reference)
import functools

import jax
import jax.numpy as jnp
from jax import lax
from jax.experimental import pallas as pl
from jax.experimental.pallas import tpu as pltpu
from jax.experimental.pallas import tpu_sc as plsc

N, K, D = 10000, 32, 128
B = N * K
NC, NS = 2, 16
NW = NC * NS
BLK_ROWS = 8
BLK = BLK_ROWS * K
NBLK = B // BLK
BLK_PER_W = NBLK // NW
EXTRA = NBLK % NW
NSLOT = 3


def _gather_sc(x, idx):
    mesh = plsc.VectorSubcoreMesh(core_axis_name="c", subcore_axis_name="s")

    @functools.partial(
        pl.kernel,
        mesh=mesh,
        out_type=jax.ShapeDtypeStruct((N, K * D), jnp.float32),
        scratch_types=[pltpu.VMEM((BLK,), jnp.int32) for _ in range(NSLOT)]
        + [pltpu.VMEM((BLK, D), jnp.float32) for _ in range(NSLOT)]
        + [pltpu.SemaphoreType.DMA for _ in range(3 * NSLOT)],
    )
    def k(x_hbm, idx_hbm, out_hbm, *scratch):
        ibufs = scratch[:NSLOT]
        gbufs = scratch[NSLOT:2 * NSLOT]
        isems = scratch[2 * NSLOT:3 * NSLOT]
        gsems = scratch[3 * NSLOT:4 * NSLOT]
        ssems = scratch[4 * NSLOT:]

        wid = lax.axis_index("s") * NC + lax.axis_index("c")
        b0 = wid * BLK_PER_W + jnp.minimum(wid, EXTRA)
        nblk = BLK_PER_W + jnp.where(wid < EXTRA, 1, 0)

        def i_copy(j, s):
            off = pl.multiple_of((b0 + j) * BLK, 8)
            return pltpu.make_async_copy(
                idx_hbm.at[pl.ds(off, BLK)], ibufs[s], isems[s])

        def g_copy(j, s):
            return pltpu.make_async_copy(
                x_hbm.at[ibufs[s]], gbufs[s], gsems[s])

        def s_copy(j, s):
            return pltpu.make_async_copy(
                gbufs[s].reshape(BLK_ROWS, K * D),
                out_hbm.at[pl.ds((b0 + j) * BLK_ROWS, BLK_ROWS), :],
                ssems[s])

        i_copy(0, 0).start()

        @pl.when(nblk > 1)
        def _():
            i_copy(1, 1).start()

        i_copy(0, 0).wait()
        g_copy(0, 0).start()

        def body(j, carry):
            for s in range(NSLOT):
                jj = j * NSLOT + s
                s1 = (s + 1) % NSLOT
                s2 = (s + 2) % NSLOT

                @pl.when(jj < nblk)
                def _():
                    @pl.when(jj + 2 < nblk)
                    def _():
                        i_copy(jj + 2, s2).start()

                    @pl.when(jj >= 2)
                    def _():
                        s_copy(jj - 2, s1).wait()

                    @pl.when(jj + 1 < nblk)
                    def _():
                        i_copy(jj + 1, s1).wait()
                        g_copy(jj + 1, s1).start()

                    g_copy(jj, s).wait()
                    s_copy(jj, s).start()

            return carry

        lax.fori_loop(0, (BLK_PER_W + 1 + NSLOT - 1) // NSLOT, body, 0)

        @pl.when(wid < EXTRA)
        def _():
            s_copy(38, 38 % NSLOT).wait()
            s_copy(39, 39 % NSLOT).wait()

        @pl.when(wid >= EXTRA)
        def _():
            s_copy(37, 37 % NSLOT).wait()
            s_copy(38, 38 % NSLOT).wait()

    return k(x, idx)


def kernel(X, G):
    return _gather_sc(X, G.reshape(-1).astype(jnp.int32))


# speedup vs baseline: 1.0009x; 1.0009x over previous
import functools

import jax
import jax.numpy as jnp
from jax import lax
from jax.experimental import pallas as pl
from jax.experimental.pallas import tpu as pltpu
from jax.experimental.pallas import tpu_sc as plsc

N, K, D = 10000, 32, 128
B = N * K
NC, NS = 2, 16
NW = NC * NS
BLK_ROWS = 8
BLK = BLK_ROWS * K
NBLK = B // BLK
BLK_PER_W = NBLK // NW
EXTRA = NBLK % NW
NSLOT = 3


def _gather_sc(x, idx):
    mesh = plsc.VectorSubcoreMesh(core_axis_name="c", subcore_axis_name="s")

    @functools.partial(
        pl.kernel,
        mesh=mesh,
        out_type=jax.ShapeDtypeStruct((N, K * D), jnp.float32),
        scratch_types=[pltpu.VMEM((BLK,), jnp.int32) for _ in range(NSLOT)]
        + [pltpu.VMEM((BLK, D), jnp.float32) for _ in range(NSLOT)]
        + [pltpu.SemaphoreType.DMA for _ in range(3 * NSLOT)],
    )
    def k(x_hbm, idx_hbm, out_hbm, *scratch):
        ibufs = scratch[:NSLOT]
        gbufs = scratch[NSLOT:2 * NSLOT]
        isems = scratch[2 * NSLOT:3 * NSLOT]
        gsems = scratch[3 * NSLOT:4 * NSLOT]
        ssems = scratch[4 * NSLOT:]

        wid = lax.axis_index("s") * NC + lax.axis_index("c")
        b0 = wid * BLK_PER_W + jnp.minimum(wid, EXTRA)
        nblk = BLK_PER_W + jnp.where(wid < EXTRA, 1, 0)

        def i_copy(j, s):
            off = pl.multiple_of((b0 + j) * BLK, 8)
            return pltpu.make_async_copy(
                idx_hbm.at[pl.ds(off, BLK)], ibufs[s], isems[s])

        def g_parts(j, s):
            return [
                pltpu.make_async_copy(
                    x_hbm.at[ibufs[s].at[pl.ds(h * 128, 128)]],
                    gbufs[s].at[pl.ds(h * 128, 128), :],
                    gsems[s])
                for h in range(2)
            ]

        def g_start(j, s):
            for c in g_parts(j, s):
                c.start()

        def g_wait(j, s):
            for c in g_parts(j, s):
                c.wait()

        def s_copy(j, s):
            return pltpu.make_async_copy(
                gbufs[s].reshape(BLK_ROWS, K * D),
                out_hbm.at[pl.ds((b0 + j) * BLK_ROWS, BLK_ROWS), :],
                ssems[s])

        i_copy(0, 0).start()

        @pl.when(nblk > 1)
        def _():
            i_copy(1, 1).start()

        i_copy(0, 0).wait()
        g_start(0, 0)

        def body(j, carry):
            for s in range(NSLOT):
                jj = j * NSLOT + s
                s1 = (s + 1) % NSLOT
                s2 = (s + 2) % NSLOT

                @pl.when(jj < nblk)
                def _():
                    @pl.when(jj + 2 < nblk)
                    def _():
                        i_copy(jj + 2, s2).start()

                    @pl.when(jj >= 2)
                    def _():
                        s_copy(jj - 2, s1).wait()

                    @pl.when(jj + 1 < nblk)
                    def _():
                        i_copy(jj + 1, s1).wait()
                        g_start(jj + 1, s1)

                    g_wait(jj, s)
                    s_copy(jj, s).start()

            return carry

        lax.fori_loop(0, (BLK_PER_W + 1 + NSLOT - 1) // NSLOT, body, 0)

        @pl.when(wid < EXTRA)
        def _():
            s_copy(38, 38 % NSLOT).wait()
            s_copy(39, 39 % NSLOT).wait()

        @pl.when(wid >= EXTRA)
        def _():
            s_copy(37, 37 % NSLOT).wait()
            s_copy(38, 38 % NSLOT).wait()

    return k(x, idx)


def kernel(X, G):
    return _gather_sc(X, G.reshape(-1).astype(jnp.int32))
